# Initial kernel scaffold; baseline (speedup 1.0000x reference)
import jax
import jax.numpy as jnp
from jax.experimental import pallas as pl


def kernel(inputs, table):
    raise NotImplementedError("write your pallas kernel here")


# speedup vs baseline: 1.0948x; 1.0948x over previous
import functools

import jax
import jax.numpy as jnp
from jax import lax
from jax.experimental import pallas as pl
from jax.experimental.pallas import tpu as pltpu
from jax.experimental.pallas import tpu_sc as plsc

NUM_EMB = 1000000
DIM = 32
BATCH = 16384
HIST = 50
B = BATCH * HIST

NC = 2
NS = 16
NW = NC * NS
B_PER_W = B // NW
CHUNK = 1024
N_CHUNKS = B_PER_W // CHUNK

_mesh = plsc.VectorSubcoreMesh(core_axis_name="c", subcore_axis_name="s")


@functools.partial(
    pl.kernel,
    mesh=_mesh,
    compiler_params=pltpu.CompilerParams(use_tc_tiling_on_sc=False),
    out_type=jax.ShapeDtypeStruct((B, DIM), jnp.float32),
    scratch_types=[
        pltpu.VMEM((CHUNK,), jnp.int32),
        pltpu.VMEM((CHUNK, DIM), jnp.float32),
        pltpu.SemaphoreType.DMA,
    ],
)
def _gather_kernel(idx_hbm, table_hbm, out_hbm, idx_v, rows_v, sem):
    wid = lax.axis_index("s") * NC + lax.axis_index("c")
    base = wid * B_PER_W

    def chunk_body(g, carry):
        off = base + g * CHUNK
        pltpu.sync_copy(idx_hbm.at[pl.ds(off, CHUNK)], idx_v)
        pltpu.async_copy(table_hbm.at[idx_v], rows_v, sem).wait()
        pltpu.sync_copy(rows_v, out_hbm.at[pl.ds(off, CHUNK)])
        return carry

    lax.fori_loop(0, N_CHUNKS, chunk_body, 0)


def kernel(inputs, table):
    idx = inputs.reshape(-1).astype(jnp.int32)
    out = _gather_kernel(idx, table)
    return out.reshape(BATCH, HIST, DIM)


# speedup vs baseline: 1.1138x; 1.0174x over previous
import functools

import jax
import jax.numpy as jnp
from jax import lax
from jax.experimental import pallas as pl
from jax.experimental.pallas import tpu as pltpu
from jax.experimental.pallas import tpu_sc as plsc

NUM_EMB = 1000000
DIM = 32
BATCH = 16384
HIST = 50
B = BATCH * HIST

NC = 2
NS = 16
NW = NC * NS
B_PER_W = B // NW

CHUNK = 256
N_CHUNKS = B_PER_W // CHUNK
D = 8
LAG1 = 1
LAG2 = 4
T_STEPS = N_CHUNKS + LAG1 + LAG2
N_OUTER = -(-T_STEPS // D)

_mesh = plsc.VectorSubcoreMesh(core_axis_name="c", subcore_axis_name="s")


@functools.partial(
    pl.kernel,
    mesh=_mesh,
    compiler_params=pltpu.CompilerParams(use_tc_tiling_on_sc=False),
    out_type=jax.ShapeDtypeStruct((B, DIM), jnp.float32),
    scratch_types=[
        pltpu.VMEM((D, CHUNK), jnp.int32),
        pltpu.VMEM((D, CHUNK, DIM), jnp.float32),
        pltpu.SemaphoreType.DMA((D,)),
        pltpu.SemaphoreType.DMA((D,)),
        pltpu.SemaphoreType.DMA((D,)),
    ],
)
def _gather_kernel(idx_hbm, table_hbm, out_hbm, idx_v, rows_v,
                   idx_sem, g_sem, s_sem):
    wid = lax.axis_index("s") * NC + lax.axis_index("c")
    base = wid * B_PER_W

    def idx_copy(c, b):
        return pltpu.make_async_copy(
            idx_hbm.at[pl.ds(base + c * CHUNK, CHUNK)], idx_v.at[b],
            idx_sem.at[b])

    def gather_copy(b):
        return pltpu.make_async_copy(
            table_hbm.at[idx_v.at[b]], rows_v.at[b], g_sem.at[b])

    def store_copy(c, b):
        return pltpu.make_async_copy(
            rows_v.at[b], out_hbm.at[pl.ds(base + c * CHUNK, CHUNK)],
            s_sem.at[b])

    def outer(o, carry):
        for b in range(D):
            t = o * D + b
            c_a = t
            c_b = t - LAG1
            c_c = t - LAG1 - LAG2
            bb = (b - LAG1) % D
            bc = (b - LAG1 - LAG2) % D

            @pl.when(jnp.logical_and(c_a >= D, c_a < N_CHUNKS))
            def _():
                store_copy(c_a - D, b).wait()

            @pl.when(c_a < N_CHUNKS)
            def _():
                idx_copy(c_a, b).start()

            @pl.when(jnp.logical_and(c_b >= 0, c_b < N_CHUNKS))
            def _():
                idx_copy(c_b, bb).wait()
                gather_copy(bb).start()

            @pl.when(jnp.logical_and(c_c >= 0, c_c < N_CHUNKS))
            def _():
                gather_copy(bc).wait()
                store_copy(c_c, bc).start()
        return carry

    lax.fori_loop(0, N_OUTER, outer, 0)

    for b in range(D):
        c = N_CHUNKS - D + b
        store_copy(c, c % D).wait()


def kernel(inputs, table):
    idx = inputs.reshape(-1).astype(jnp.int32)
    out = _gather_kernel(idx, table)
    return out.reshape(BATCH, HIST, DIM)


# speedup vs baseline: 1.8189x; 1.6331x over previous
import functools

import jax
import jax.numpy as jnp
from jax import lax
from jax.experimental import pallas as pl
from jax.experimental.pallas import tpu as pltpu
from jax.experimental.pallas import tpu_sc as plsc

NUM_EMB = 1000000
DIM = 32
BATCH = 16384
HIST = 50
B = BATCH * HIST

NC = 2
NS = 16
NW = NC * NS
B_PER_W = BATCH // NW

CB = 4
CHUNK = CB * HIST
N_CHUNKS = B_PER_W // CB
D = 8
LAG1 = 1
LAG2 = 4
T_STEPS = N_CHUNKS + LAG1 + LAG2
N_OUTER = -(-T_STEPS // D)

_mesh = plsc.VectorSubcoreMesh(core_axis_name="c", subcore_axis_name="s")


@functools.partial(
    pl.kernel,
    mesh=_mesh,
    compiler_params=pltpu.CompilerParams(use_tc_tiling_on_sc=False),
    out_type=jax.ShapeDtypeStruct((BATCH, HIST, DIM), jnp.float32),
    scratch_types=[
        pltpu.VMEM((D, CHUNK), jnp.int32),
        pltpu.VMEM((D, CHUNK, DIM), jnp.float32),
        pltpu.SemaphoreType.DMA((D,)),
        pltpu.SemaphoreType.DMA((D,)),
        pltpu.SemaphoreType.DMA((D,)),
    ],
)
def _gather_kernel(idx_hbm, table_hbm, out_hbm, idx_v, rows_v,
                   idx_sem, g_sem, s_sem):
    wid = lax.axis_index("s") * NC + lax.axis_index("c")
    base_b = wid * B_PER_W

    def idx_copy(c, b):
        return pltpu.make_async_copy(
            idx_hbm.at[pl.ds((base_b + c * CB) * HIST, CHUNK)], idx_v.at[b],
            idx_sem.at[b])

    def gather_copy(b):
        return pltpu.make_async_copy(
            table_hbm.at[idx_v.at[b]], rows_v.at[b], g_sem.at[b])

    def store_copy(c, b, k):
        return pltpu.make_async_copy(
            rows_v.at[b, pl.ds(k * HIST, HIST)],
            out_hbm.at[base_b + c * CB + k], s_sem.at[b])

    def outer(o, carry):
        for b in range(D):
            t = o * D + b
            c_a = t
            c_b = t - LAG1
            c_c = t - LAG1 - LAG2
            bb = (b - LAG1) % D
            bc = (b - LAG1 - LAG2) % D

            @pl.when(jnp.logical_and(c_a >= D, c_a < N_CHUNKS))
            def _():
                for k in range(CB):
                    store_copy(c_a - D, b, k).wait()

            @pl.when(c_a < N_CHUNKS)
            def _():
                idx_copy(c_a, b).start()

            @pl.when(jnp.logical_and(c_b >= 0, c_b < N_CHUNKS))
            def _():
                idx_copy(c_b, bb).wait()
                gather_copy(bb).start()

            @pl.when(jnp.logical_and(c_c >= 0, c_c < N_CHUNKS))
            def _():
                gather_copy(bc).wait()
                for k in range(CB):
                    store_copy(c_c, bc, k).start()
        return carry

    lax.fori_loop(0, N_OUTER, outer, 0)

    for b in range(D):
        c = N_CHUNKS - D + b
        for k in range(CB):
            store_copy(c, c % D, k).wait()


def kernel(inputs, table):
    idx = inputs.reshape(-1).astype(jnp.int32)
    return _gather_kernel(idx, table)
